# baseline (speedup 1.0000x reference)
import functools

import jax
import jax.numpy as jnp
from jax import lax
from jax.experimental import pallas as pl
from jax.experimental.pallas import tpu as pltpu
from jax.experimental.pallas import tpu_sc as plsc

HIDDEN = 64
N_TOKENS = 800
N_TOKENS_PAD = 1024
VB = 2048

_NC, _NS = 2, 16
_NW = _NC * _NS
_B_PER_W = N_TOKENS_PAD // _NW


@functools.cache
def _make_sc_gather():
    @functools.partial(
        pl.kernel,
        mesh=plsc.VectorSubcoreMesh(core_axis_name="c", subcore_axis_name="s"),
        out_type=jax.ShapeDtypeStruct((N_TOKENS_PAD, HIDDEN), jnp.float32),
        scratch_types=[
            pltpu.VMEM((_B_PER_W,), jnp.int32),
            pltpu.VMEM((_B_PER_W, HIDDEN), jnp.float32),
            pltpu.SemaphoreType.DMA,
        ],
        compiler_params=pltpu.CompilerParams(use_tc_tiling_on_sc=False),
    )
    def _sc_gather(idx_hbm, table_hbm, out_hbm, idx_v, rows_v, sem):
        wid = lax.axis_index("s") * _NC + lax.axis_index("c")
        base = wid * _B_PER_W
        pltpu.sync_copy(idx_hbm.at[pl.ds(base, _B_PER_W)], idx_v)
        pltpu.async_copy(table_hbm.at[idx_v], rows_v, sem).wait()
        pltpu.sync_copy(rows_v, out_hbm.at[pl.ds(base, _B_PER_W)])

    return _sc_gather


def _head_body(h_ref, w_ref, out_ref):
    out_ref[...] = lax.dot_general(
        h_ref[...], w_ref[...],
        (((1,), (1,)), ((), ())),
        preferred_element_type=jnp.float32,
    )


def kernel(input_ids, attention_mask, embed, lm_head_w):
    del attention_mask
    B, T = input_ids.shape
    V = lm_head_w.shape[0]

    ids = jnp.reshape(input_ids, (-1,)).astype(jnp.int32)
    ids = jnp.pad(ids, (0, N_TOKENS_PAD - N_TOKENS))

    h = _make_sc_gather()(ids, embed)[:N_TOKENS]

    n_vb = pl.cdiv(V, VB)
    logits = pl.pallas_call(
        _head_body,
        grid=(n_vb,),
        in_specs=[
            pl.BlockSpec((N_TOKENS, HIDDEN), lambda i: (0, 0)),
            pl.BlockSpec((VB, HIDDEN), lambda i: (i, 0)),
        ],
        out_specs=pl.BlockSpec((N_TOKENS, VB), lambda i: (0, i)),
        out_shape=jax.ShapeDtypeStruct((N_TOKENS, V), jnp.float32),
    )(h, lm_head_w)

    return jnp.reshape(logits, (B, T, V))
